# baseline (speedup 1.0000x reference)
import functools
import jax
import jax.numpy as jnp
from jax import lax
from jax.experimental import pallas as pl
from jax.experimental.pallas import tpu as pltpu
from jax.experimental.pallas import tpu_sc as plsc

_NW = 32
_CH = 32


def kernel(x, emb_weight):
    B, L, D = x.shape
    rows = L // _NW
    nch = rows // _CH
    mesh = plsc.VectorSubcoreMesh(core_axis_name="c", subcore_axis_name="s")

    @functools.partial(
        pl.kernel,
        mesh=mesh,
        out_type=jax.ShapeDtypeStruct((B, L, D), emb_weight.dtype),
        scratch_types=[
            pltpu.VMEM((2, _CH, D), emb_weight.dtype),
            pltpu.SemaphoreType.DMA,
            pltpu.SemaphoreType.DMA,
            pltpu.SemaphoreType.DMA,
            pltpu.SemaphoreType.DMA,
        ],
    )
    def sc_copy(emb_hbm, out_hbm, buf, isem0, isem1, osem0, osem1):
        wid = lax.axis_index("s") * 2 + lax.axis_index("c")
        base = wid * rows
        isems = (isem0, isem1)
        osems = (osem0, osem1)

        def read(j, slot):
            return pltpu.make_async_copy(
                emb_hbm.at[pl.ds(base + j * _CH, _CH), :], buf.at[slot], isems[slot]
            )

        def write(j, slot, b):
            return pltpu.make_async_copy(
                buf.at[slot], out_hbm.at[b, pl.ds(base + j * _CH, _CH), :], osems[slot]
            )

        read(0, 0).start()

        @pl.loop(0, nch, step=2)
        def _(j):
            for s in range(2):
                cur = j + s

                @pl.when(cur >= 1)
                def _():
                    for b in range(B):
                        write(cur - 1, 1 - s, b).wait()

                @pl.when(cur + 1 < nch)
                def _():
                    read(cur + 1, 1 - s).start()

                read(cur, s).wait()
                for b in range(B):
                    write(cur, s, b).start()

        for b in range(B):
            write(nch - 1, (nch - 1) % 2, b).wait()

    return sc_copy(emb_weight)


# speedup vs baseline: 1.0111x; 1.0111x over previous
import functools
import jax
import jax.numpy as jnp
from jax import lax
from jax.experimental import pallas as pl
from jax.experimental.pallas import tpu as pltpu
from jax.experimental.pallas import tpu_sc as plsc

_NW = 32
_CH = 64


def kernel(x, emb_weight):
    B, L, D = x.shape
    rows = L // _NW
    nch = rows // _CH
    mesh = plsc.VectorSubcoreMesh(core_axis_name="c", subcore_axis_name="s")

    @functools.partial(
        pl.kernel,
        mesh=mesh,
        out_type=jax.ShapeDtypeStruct((B, L, D), emb_weight.dtype),
        scratch_types=[
            pltpu.VMEM((_CH, D), emb_weight.dtype),
            pltpu.SemaphoreType.DMA,
            pltpu.SemaphoreType.DMA,
        ],
    )
    def sc_copy(emb_hbm, out_hbm, buf, isem, osem):
        wid = lax.axis_index("s") * 2 + lax.axis_index("c")
        base = wid * rows

        @pl.loop(0, nch)
        def _(j):
            off = base + j * _CH
            pltpu.async_copy(emb_hbm.at[pl.ds(off, _CH), :], buf, isem).wait()
            for b in range(B):
                pltpu.async_copy(buf, out_hbm.at[b, pl.ds(off, _CH), :], osem)
            for b in range(B):
                pltpu.make_async_copy(
                    buf, out_hbm.at[b, pl.ds(off, _CH), :], osem
                ).wait()

    return sc_copy(emb_weight)
